# baseline (speedup 1.0000x reference)
import functools

import jax
import jax.numpy as jnp
from jax import lax
from jax.experimental import pallas as pl
from jax.experimental.pallas import tpu as pltpu
from jax.experimental.pallas import tpu_sc as plsc

_NC = 2
_NS = 16
_NW = _NC * _NS
_LANES = 16

_D = 32
_K = 20
_C = 16
_IDX_CHUNK = 128
_SR = 128


def _relayout_body(ctab_hbm, xtab_hbm, ctail_hbm, xtail_hbm,
                   cout_hbm, xout_hbm,
                   tin0, tin1, tout0, tout1, sem0, sem1, osem, *, v):
    wid = lax.axis_index("s") * _NC + lax.axis_index("c")
    lane = lax.iota(jnp.int32, _LANES)
    lane16 = lane + 16
    nfull = v // 512
    tailn = (v - nfull * 512) // 4
    npair = (nfull // _NW + 1 + 1) // 2

    for tab_hbm, tail_hbm, out_hbm in ((ctab_hbm, ctail_hbm, cout_hbm),
                                       (xtab_hbm, xtail_hbm, xout_hbm)):
        def issue_in(blk, tin, sem, tab_hbm=tab_hbm):
            @pl.when(blk < nfull)
            def _():
                pltpu.async_copy(
                    tab_hbm.at[pl.ds(0, 32), pl.ds(blk * 512, 512)],
                    tin, sem)

        def wait_in(blk, tin, sem, tab_hbm=tab_hbm):
            @pl.when(blk < nfull)
            def _():
                pltpu.make_async_copy(
                    tab_hbm.at[pl.ds(0, 32), pl.ds(blk * 512, 512)],
                    tin, sem).wait()

        def do_block(blk, tin, tout, tail_hbm=tail_hbm, out_hbm=out_hbm):
            @pl.when(blk < nfull)
            def _():
                lane4 = lane >> 2
                lanem32 = (lane & 3) * 32

                @plsc.parallel_loop(0, 32, step=1, unroll=4)
                def dstep(d):
                    colv = lanem32 + d
                    for cgb in range(8):
                        xs = [tin[d, pl.ds((cgb * 4 + j) * 16, 16)]
                              for j in range(4)]
                        for j in range(4):
                            plsc.store_scatter(
                                tout, [lane4 + ((cgb * 4 + j) * 4), colv],
                                xs[j])

                pltpu.async_copy(tout, out_hbm.at[pl.ds(blk * 128, 128)],
                                 osem)

            @pl.when((blk == nfull) & (tailn > 0))
            def _():
                pltpu.sync_copy(tail_hbm, tout.at[pl.ds(0, tailn)])
                pltpu.async_copy(tout.at[pl.ds(0, tailn)],
                                 out_hbm.at[pl.ds(nfull * 128, tailn)], osem)

        def wait_out(blk, tout, out_hbm=out_hbm):
            @pl.when(blk < nfull)
            def _():
                pltpu.make_async_copy(
                    tout, out_hbm.at[pl.ds(0, 128)], osem).wait()

            @pl.when((blk == nfull) & (tailn > 0))
            def _():
                pltpu.make_async_copy(
                    tout.at[pl.ds(0, tailn)],
                    out_hbm.at[pl.ds(0, tailn)], osem).wait()

        issue_in(wid, tin0, sem0)

        def pair(j, _):
            blk0 = wid + (2 * j) * _NW
            blk1 = blk0 + _NW
            wait_in(blk0, tin0, sem0)
            issue_in(blk1, tin1, sem1)
            do_block(blk0, tin0, tout0)
            wait_in(blk1, tin1, sem1)
            issue_in(blk1 + _NW, tin0, sem0)
            do_block(blk1, tin1, tout1)
            wait_out(blk0, tout0)
            wait_out(blk1, tout1)
            return 0

        lax.fori_loop(0, npair, pair, 0)


def _sc_relayout(ctab_t, xtab_t, ctail, xtail):
    d, v = ctab_t.shape
    mesh = plsc.VectorSubcoreMesh(core_axis_name="c", subcore_axis_name="s")
    fn = pl.kernel(
        functools.partial(_relayout_body, v=v),
        out_type=(
            jax.ShapeDtypeStruct((v // 4, 128), jnp.float32),
            jax.ShapeDtypeStruct((v // 4, 128), jnp.float32),
        ),
        mesh=mesh,
        scratch_types=[
            pltpu.VMEM((32, 512), jnp.float32),
            pltpu.VMEM((32, 512), jnp.float32),
            pltpu.VMEM((128, 128), jnp.float32),
            pltpu.VMEM((128, 128), jnp.float32),
            pltpu.SemaphoreType.DMA,
            pltpu.SemaphoreType.DMA,
            pltpu.SemaphoreType.DMA,
        ],
        compiler_params=pltpu.CompilerParams(
            needs_layout_passes=False, use_tc_tiling_on_sc=True),
        name="w2v_sc_relayout",
    )
    return fn(ctab_t, xtab_t, ctail, xtail)


def _issue_chunk(co, cemb_hbm, xemb_hbm, csidx, xsidx, nsidx, crows, xrows,
                 nrows, sem):
    pltpu.async_copy(cemb_hbm.at[csidx.at[pl.ds(co, _C)]], crows, sem)
    pltpu.async_copy(xemb_hbm.at[xsidx.at[pl.ds(co, _C)]], xrows, sem)
    nr = _C * _K
    for j in range(0, nr, _IDX_CHUNK):
        w = min(_IDX_CHUNK, nr - j)
        pltpu.async_copy(
            xemb_hbm.at[nsidx.at[pl.ds(co * _K + j, w)]],
            nrows.at[pl.ds(j, w)], sem)


def _drain_chunk(co, cemb_hbm, xemb_hbm, csidx, xsidx, nsidx, crows, xrows,
                 nrows, sem):
    pltpu.make_async_copy(cemb_hbm.at[csidx.at[pl.ds(co, _C)]], crows,
                          sem).wait()
    pltpu.make_async_copy(xemb_hbm.at[xsidx.at[pl.ds(co, _C)]], xrows,
                          sem).wait()
    nr = _C * _K
    for j in range(0, nr, _IDX_CHUNK):
        w = min(_IDX_CHUNK, nr - j)
        pltpu.make_async_copy(
            xemb_hbm.at[nsidx.at[pl.ds(co * _K + j, w)]],
            nrows.at[pl.ds(j, w)], sem).wait()


def _compute_chunk(co, lane, cidx, xidx, nidx, crows, xrows, nrows,
                   posb, negb):
    rows = lane
    nbase = (co + rows) * _K
    cq = (cidx[pl.ds(co, _C)] & 3) * _D
    xq = (xidx[pl.ds(co, _C)] & 3) * _D

    for h in range(2):
        cregs = [
            plsc.load_gather(crows, [rows, cq + (h * 16 + t)])
            for t in range(16)
        ]

        pparts = []
        for q in range(4):
            acc = None
            for t in range(q * 4, q * 4 + 4):
                xv = plsc.load_gather(xrows, [rows, xq + (h * 16 + t)])
                term = cregs[t] * xv
                acc = term if acc is None else acc + term
            pparts.append(acc)
        accp = (pparts[0] + pparts[1]) + (pparts[2] + pparts[3])
        if h == 0:
            plsc.store_scatter(posb, [co + rows], accp)
        else:
            plsc.addupdate_scatter(posb, [co + rows], accp)

        @plsc.parallel_loop(0, _K, step=1, unroll=2)
        def kstep(k, h=h, cregs=cregs):
            nrow = nbase + k
            nq = (plsc.load_gather(nidx, [nrow]) & 3) * _D
            parts = []
            for q in range(4):
                acc = None
                for t in range(q * 4, q * 4 + 4):
                    nv = plsc.load_gather(nrows, [(rows * _K) + k,
                                                  nq + (h * 16 + t)])
                    term = cregs[t] * nv
                    acc = term if acc is None else acc + term
                parts.append(acc)
            accn = (parts[0] + parts[1]) + (parts[2] + parts[3])
            if h == 0:
                plsc.store_scatter(negb, [nrow], accn)
            else:
                plsc.addupdate_scatter(negb, [nrow], accn)


def _sc_scores_body(c_hbm, x_hbm, n_hbm, cs_hbm, xs_hbm, ns_hbm,
                    cemb_hbm, xemb_hbm,
                    pos_hbm, negs_hbm,
                    cidx, xidx, nidx, csidx, xsidx, nsidx,
                    crows0, xrows0, nrows0,
                    crows1, xrows1, nrows1, posb, negb, sem0, sem1,
                    *, pb):
    wid = lax.axis_index("s") * _NC + lax.axis_index("c")
    base = wid * pb

    pltpu.sync_copy(c_hbm.at[pl.ds(base, pb)], cidx)
    pltpu.sync_copy(x_hbm.at[pl.ds(base, pb)], xidx)
    pltpu.sync_copy(n_hbm.at[pl.ds(base * _K, pb * _K)], nidx)
    pltpu.sync_copy(cs_hbm.at[pl.ds(base, pb)], csidx)
    pltpu.sync_copy(xs_hbm.at[pl.ds(base, pb)], xsidx)
    pltpu.sync_copy(ns_hbm.at[pl.ds(base * _K, pb * _K)], nsidx)

    lane = lax.iota(jnp.int32, _LANES)
    bufs = (
        (crows0, xrows0, nrows0, sem0),
        (crows1, xrows1, nrows1, sem1),
    )
    tbl = (cemb_hbm, xemb_hbm, csidx, xsidx, nsidx)
    nchunks = pb // _C

    _issue_chunk(0, *tbl, *bufs[0])

    def pair(cc, _):
        co0 = (2 * cc) * _C
        co1 = co0 + _C
        _drain_chunk(co0, *tbl, *bufs[0])
        _issue_chunk(co1, *tbl, *bufs[1])
        _compute_chunk(co0, lane, cidx, xidx, nidx,
                       bufs[0][0], bufs[0][1], bufs[0][2], posb, negb)

        _drain_chunk(co1, *tbl, *bufs[1])

        @pl.when(cc < (nchunks // 2) - 1)
        def _():
            _issue_chunk(co1 + _C, *tbl, *bufs[0])

        _compute_chunk(co1, lane, cidx, xidx, nidx,
                       bufs[1][0], bufs[1][1], bufs[1][2], posb, negb)
        return 0

    lax.fori_loop(0, nchunks // 2, pair, 0)
    pltpu.sync_copy(posb, pos_hbm.at[pl.ds(base, pb)])
    pltpu.sync_copy(negb, negs_hbm.at[pl.ds(base * _K, pb * _K)])


def _sc_scores(center, context, neg_flat, center_s, context_s, neg_s,
               center_emb4, context_emb4):
    b = center.shape[0]
    pb = b // _NW
    mesh = plsc.VectorSubcoreMesh(core_axis_name="c", subcore_axis_name="s")
    fn = pl.kernel(
        functools.partial(_sc_scores_body, pb=pb),
        out_type=(
            jax.ShapeDtypeStruct((b,), jnp.float32),
            jax.ShapeDtypeStruct((b * _K,), jnp.float32),
        ),
        mesh=mesh,
        scratch_types=[
            pltpu.VMEM((pb,), jnp.int32),
            pltpu.VMEM((pb,), jnp.int32),
            pltpu.VMEM((pb * _K,), jnp.int32),
            pltpu.VMEM((pb,), jnp.int32),
            pltpu.VMEM((pb,), jnp.int32),
            pltpu.VMEM((pb * _K,), jnp.int32),
            pltpu.VMEM((_C, _SR), jnp.float32),
            pltpu.VMEM((_C, _SR), jnp.float32),
            pltpu.VMEM((_C * _K, _SR), jnp.float32),
            pltpu.VMEM((_C, _SR), jnp.float32),
            pltpu.VMEM((_C, _SR), jnp.float32),
            pltpu.VMEM((_C * _K, _SR), jnp.float32),
            pltpu.VMEM((pb,), jnp.float32),
            pltpu.VMEM((pb * _K,), jnp.float32),
            pltpu.SemaphoreType.DMA,
            pltpu.SemaphoreType.DMA,
        ],
        compiler_params=pltpu.CompilerParams(
            needs_layout_passes=False, use_tc_tiling_on_sc=True),
        name="w2v_sc_scores",
    )
    return fn(center, context, neg_flat, center_s, context_s, neg_s,
              center_emb4, context_emb4)


def _loss_body(pos_ref, neg_ref, out_ref, *, b, k):
    p = pos_ref[...]
    n = neg_ref[...]
    sp = 1.0 / (1.0 + jnp.exp(-p))
    sn = 1.0 / (1.0 + jnp.exp(-n))
    lp = jnp.log(sp + 1e-9)
    ln = jnp.log(1.0 - sn + 1e-9)
    loss = -(jnp.sum(lp) / b) - (jnp.sum(ln) / (b * k))
    out_ref[...] = jnp.full((1, 1), loss, jnp.float32)


def _tc_loss(pos2d, neg2d, b, k):
    fn = pl.pallas_call(
        functools.partial(_loss_body, b=b, k=k),
        out_shape=jax.ShapeDtypeStruct((1, 1), jnp.float32),
    )
    return fn(pos2d, neg2d)


def kernel(center, context, negative_samples, center_emb, context_emb):
    b = center.shape[0]
    k = negative_samples.shape[1]
    v = center_emb.shape[0]
    neg_flat = negative_samples.reshape(b * k)
    vtail = (v // 512) * 512
    cemb4, xemb4 = _sc_relayout(
        center_emb.T, context_emb.T,
        center_emb[vtail:].reshape((v - vtail) // 4, 128),
        context_emb[vtail:].reshape((v - vtail) // 4, 128))
    pos, negs = _sc_scores(center, context, neg_flat,
                           center >> 2, context >> 2, neg_flat >> 2,
                           cemb4, xemb4)
    pos2d = pos.reshape(b // 128, 128)
    neg2d = negs.reshape((b * k) // 128, 128)
    loss = _tc_loss(pos2d, neg2d, b, k)
    return loss[0, 0]


# speedup vs baseline: 1.1906x; 1.1906x over previous
import functools

import jax
import jax.numpy as jnp
from jax import lax
from jax.experimental import pallas as pl
from jax.experimental.pallas import tpu as pltpu
from jax.experimental.pallas import tpu_sc as plsc

_NC = 2
_NS = 16
_NW = _NC * _NS
_LANES = 16

_D = 32
_K = 20
_C = 64
_IDX_CHUNK = 128


def _issue_chunk(co, cemb_hbm, xemb_hbm, cidx, xidx, nidx, crows, xrows,
                 nrows, sem):
    pltpu.async_copy(cemb_hbm.at[cidx.at[pl.ds(co, _C)]], crows, sem)
    pltpu.async_copy(xemb_hbm.at[xidx.at[pl.ds(co, _C)]], xrows, sem)
    for j in range(_C * _K // _IDX_CHUNK):
        pltpu.async_copy(
            xemb_hbm.at[nidx.at[pl.ds(co * _K + j * _IDX_CHUNK, _IDX_CHUNK)]],
            nrows.at[pl.ds(j * _IDX_CHUNK, _IDX_CHUNK)], sem)


def _drain_chunk(co, cemb_hbm, xemb_hbm, cidx, xidx, nidx, crows, xrows,
                 nrows, sem):
    pltpu.make_async_copy(cemb_hbm.at[cidx.at[pl.ds(co, _C)]], crows,
                          sem).wait()
    pltpu.make_async_copy(xemb_hbm.at[xidx.at[pl.ds(co, _C)]], xrows,
                          sem).wait()
    for j in range(_C * _K // _IDX_CHUNK):
        pltpu.make_async_copy(
            xemb_hbm.at[nidx.at[pl.ds(co * _K + j * _IDX_CHUNK, _IDX_CHUNK)]],
            nrows.at[pl.ds(j * _IDX_CHUNK, _IDX_CHUNK)], sem).wait()


def _compute_chunk(lane, crows, xrows, nrows, posb, negb):
    def group(g, _):
        rows = g * _LANES + lane
        nbase = rows * _K

        def pstep(d, acc):
            dcol = jnp.full((_LANES,), d, jnp.int32)
            cvec = plsc.load_gather(crows, [rows, dcol])
            xvec = plsc.load_gather(xrows, [rows, dcol])
            return acc + cvec * xvec

        accp = lax.fori_loop(0, _D, pstep, jnp.zeros((_LANES,), jnp.float32))
        plsc.store_scatter(posb, [rows], accp)

        for h in range(2):
            cregs = [
                plsc.load_gather(
                    crows, [rows, jnp.full((_LANES,), h * 16 + t, jnp.int32)])
                for t in range(16)
            ]

            def kstep(k, _, h=h, cregs=cregs):
                nrow = nbase + k
                parts = []
                for q in range(4):
                    acc = None
                    for t in range(q * 4, q * 4 + 4):
                        dcol = jnp.full((_LANES,), h * 16 + t, jnp.int32)
                        nv = plsc.load_gather(nrows, [nrow, dcol])
                        term = cregs[t] * nv
                        acc = term if acc is None else acc + term
                    parts.append(acc)
                accn = (parts[0] + parts[1]) + (parts[2] + parts[3])
                if h == 0:
                    plsc.store_scatter(negb, [nrow], accn)
                else:
                    plsc.addupdate_scatter(negb, [nrow], accn)
                return 0

            lax.fori_loop(0, _K, kstep, 0)
        return 0

    lax.fori_loop(0, _C // _LANES, group, 0)


def _sc_scores_body(c_hbm, x_hbm, n_hbm, cemb_hbm, xemb_hbm,
                    pos_hbm, negs_hbm,
                    cidx, xidx, nidx, crows0, xrows0, nrows0,
                    crows1, xrows1, nrows1, posb, negb, sem0, sem1,
                    *, pb):
    wid = lax.axis_index("s") * _NC + lax.axis_index("c")
    base = wid * pb

    pltpu.sync_copy(c_hbm.at[pl.ds(base, pb)], cidx)
    pltpu.sync_copy(x_hbm.at[pl.ds(base, pb)], xidx)
    pltpu.sync_copy(n_hbm.at[pl.ds(base * _K, pb * _K)], nidx)

    lane = lax.iota(jnp.int32, _LANES)
    bufs = (
        (crows0, xrows0, nrows0, sem0),
        (crows1, xrows1, nrows1, sem1),
    )
    tbl = (cemb_hbm, xemb_hbm, cidx, xidx, nidx)
    nchunks = pb // _C

    _issue_chunk(0, *tbl, *bufs[0])

    def pair(cc, _):
        co0 = (2 * cc) * _C
        co1 = co0 + _C
        _drain_chunk(co0, *tbl, *bufs[0])
        _issue_chunk(co1, *tbl, *bufs[1])
        _compute_chunk(lane, bufs[0][0], bufs[0][1], bufs[0][2], posb, negb)
        pltpu.sync_copy(posb, pos_hbm.at[pl.ds(base + co0, _C)])
        pltpu.sync_copy(negb, negs_hbm.at[pl.ds((base + co0) * _K, _C * _K)])

        _drain_chunk(co1, *tbl, *bufs[1])

        @pl.when(cc < (nchunks // 2) - 1)
        def _():
            _issue_chunk(co1 + _C, *tbl, *bufs[0])

        _compute_chunk(lane, bufs[1][0], bufs[1][1], bufs[1][2], posb, negb)
        pltpu.sync_copy(posb, pos_hbm.at[pl.ds(base + co1, _C)])
        pltpu.sync_copy(negb, negs_hbm.at[pl.ds((base + co1) * _K, _C * _K)])
        return 0

    lax.fori_loop(0, nchunks // 2, pair, 0)


def _sc_scores(center, context, neg_flat, center_emb, context_emb):
    b = center.shape[0]
    pb = b // _NW
    mesh = plsc.VectorSubcoreMesh(core_axis_name="c", subcore_axis_name="s")
    fn = pl.kernel(
        functools.partial(_sc_scores_body, pb=pb),
        out_type=(
            jax.ShapeDtypeStruct((b,), jnp.float32),
            jax.ShapeDtypeStruct((b * _K,), jnp.float32),
        ),
        mesh=mesh,
        scratch_types=[
            pltpu.VMEM((pb,), jnp.int32),
            pltpu.VMEM((pb,), jnp.int32),
            pltpu.VMEM((pb * _K,), jnp.int32),
            pltpu.VMEM((_C, _D), jnp.float32),
            pltpu.VMEM((_C, _D), jnp.float32),
            pltpu.VMEM((_C * _K, _D), jnp.float32),
            pltpu.VMEM((_C, _D), jnp.float32),
            pltpu.VMEM((_C, _D), jnp.float32),
            pltpu.VMEM((_C * _K, _D), jnp.float32),
            pltpu.VMEM((_C,), jnp.float32),
            pltpu.VMEM((_C * _K,), jnp.float32),
            pltpu.SemaphoreType.DMA,
            pltpu.SemaphoreType.DMA,
        ],
        compiler_params=pltpu.CompilerParams(
            needs_layout_passes=False, use_tc_tiling_on_sc=False),
        name="w2v_sc_scores",
    )
    return fn(center, context, neg_flat, center_emb, context_emb)


def _loss_body(pos_ref, neg_ref, out_ref, *, b, k):
    p = pos_ref[...]
    n = neg_ref[...]
    sp = 1.0 / (1.0 + jnp.exp(-p))
    sn = 1.0 / (1.0 + jnp.exp(-n))
    lp = jnp.log(sp + 1e-9)
    ln = jnp.log(1.0 - sn + 1e-9)
    loss = -(jnp.sum(lp) / b) - (jnp.sum(ln) / (b * k))
    out_ref[...] = jnp.full((1, 1), loss, jnp.float32)


def _tc_loss(pos2d, neg2d, b, k):
    fn = pl.pallas_call(
        functools.partial(_loss_body, b=b, k=k),
        out_shape=jax.ShapeDtypeStruct((1, 1), jnp.float32),
    )
    return fn(pos2d, neg2d)


def kernel(center, context, negative_samples, center_emb, context_emb):
    b = center.shape[0]
    k = negative_samples.shape[1]
    neg_flat = negative_samples.reshape(b * k)
    pos, negs = _sc_scores(center, context, neg_flat, center_emb, context_emb)
    pos2d = pos.reshape(b // 128, 128)
    neg2d = negs.reshape((b * k) // 128, 128)
    loss = _tc_loss(pos2d, neg2d, b, k)
    return loss[0, 0]
